# baseline (speedup 1.0000x reference)
import functools

import jax
import jax.numpy as jnp
from jax import lax
from jax.experimental import pallas as pl
from jax.experimental.pallas import tpu as pltpu
from jax.experimental.pallas import tpu_sc as plsc

N_USER = 40000
N_ITEM = 10000
N_NODES = 50000
N_EDGES = 800000
LATDIM = 64
HALF = 32
IMG_DIM = 4096
TXT_DIM = 384

ROW_BLK = 1000

NUM_TILES = 16
N_EDGES_PAD = 822016
EDGES_PER_TILE = N_EDGES_PAD // NUM_TILES
CHUNK = 208
SUB = 104
NSUB = CHUNK // SUB
KCH = 13
SUPER = KCH * CHUNK
SUPERS = EDGES_PER_TILE // SUPER
N_NODES_PAD = 50048
ROWS_PER_TILE = N_NODES_PAD // NUM_TILES


def _mm_body(img_ref, txt_ref, wi_ref, wt_ref, bi_ref, bt_ref, yi_ref, yt_ref):
    yi_ref[...] = (
        jnp.dot(img_ref[...], wi_ref[...], preferred_element_type=jnp.float32)
        + bi_ref[...]
    )
    yt_ref[...] = (
        jnp.dot(txt_ref[...], wt_ref[...], preferred_element_type=jnp.float32)
        + bt_ref[...]
    )


def _dense_mm(img, txt, wi, wt, bi, bt):
    grid = (N_ITEM // ROW_BLK,)
    return pl.pallas_call(
        _mm_body,
        grid=grid,
        in_specs=[
            pl.BlockSpec((ROW_BLK, IMG_DIM), lambda i: (i, 0)),
            pl.BlockSpec((ROW_BLK, TXT_DIM), lambda i: (i, 0)),
            pl.BlockSpec((IMG_DIM, LATDIM), lambda i: (0, 0)),
            pl.BlockSpec((TXT_DIM, LATDIM), lambda i: (0, 0)),
            pl.BlockSpec((1, LATDIM), lambda i: (0, 0)),
            pl.BlockSpec((1, LATDIM), lambda i: (0, 0)),
        ],
        out_specs=[
            pl.BlockSpec((ROW_BLK, LATDIM), lambda i: (i, 0)),
            pl.BlockSpec((ROW_BLK, LATDIM), lambda i: (i, 0)),
        ],
        out_shape=[
            jax.ShapeDtypeStruct((N_ITEM, LATDIM), jnp.float32),
            jax.ShapeDtypeStruct((N_ITEM, LATDIM), jnp.float32),
        ],
    )(img, txt, wi, wt, bi, bt)


def _bn(x):
    m = jnp.mean(x, axis=0, keepdims=True)
    d = x - m
    v = jnp.mean(d * d, axis=0, keepdims=True)
    return d * lax.rsqrt(v + 1e-5)


def _leaky(x):
    return jnp.where(x >= 0, x, 0.2 * x)


def _gates_body(
    yi_ref, yt_ref, item_ref, wip_ref, bip_ref, wtp_ref, btp_ref,
    wgi_ref, bgi_ref, wgt_ref, bgt_ref, fw_ref, rs_ref, out_ref,
):
    rs = rs_ref[0, 0]
    x_img = _leaky(_bn(yi_ref[...]))
    img_feat = rs * x_img + _leaky(
        _bn(jnp.dot(x_img, wip_ref[...], preferred_element_type=jnp.float32)
            + bip_ref[...])
    )
    x_txt = _leaky(_bn(yt_ref[...]))
    txt_feat = rs * x_txt + _leaky(
        _bn(jnp.dot(x_txt, wtp_ref[...], preferred_element_type=jnp.float32)
            + btp_ref[...])
    )
    gate_img = jax.nn.sigmoid(
        _bn(jnp.dot(img_feat, wgi_ref[...], preferred_element_type=jnp.float32)
            + bgi_ref[...])
    )
    gate_txt = jax.nn.sigmoid(
        _bn(jnp.dot(txt_feat, wgt_ref[...], preferred_element_type=jnp.float32)
            + bgt_ref[...])
    )
    f0 = fw_ref[0, 0]
    f1 = fw_ref[0, 1]
    f2 = fw_ref[0, 2]
    fm = jnp.maximum(f0, jnp.maximum(f1, f2))
    e0 = jnp.exp(f0 - fm)
    e1 = jnp.exp(f1 - fm)
    e2 = jnp.exp(f2 - fm)
    es = e0 + e1 + e2
    w0 = e0 / es
    w1 = e1 / es
    w2 = e2 / es
    out_ref[...] = item_ref[...] * (w0 + w1 * gate_img + w2 * gate_txt)


def _gated_item(yi, yt, item, wip, bip, wtp, btp, wgi, bgi, wgt, bgt, fw, rs):
    return pl.pallas_call(
        _gates_body,
        out_shape=jax.ShapeDtypeStruct((N_ITEM, LATDIM), jnp.float32),
    )(yi, yt, item, wip, bip, wtp, btp, wgi, bgi, wgt, bgt, fw, rs)


def _sc_propagate(t0, t1, eidx, ev):
    mesh = plsc.VectorSubcoreMesh(core_axis_name="c", subcore_axis_name="s")

    @functools.partial(
        pl.kernel,
        out_type=[
            jax.ShapeDtypeStruct((N_NODES_PAD, HALF), jnp.float32),
            jax.ShapeDtypeStruct((N_NODES_PAD, HALF), jnp.float32),
        ],
        mesh=mesh,
        compiler_params=pltpu.CompilerParams(
            needs_layout_passes=False, use_tc_tiling_on_sc=False),
        scratch_types=[
            pltpu.VMEM((SUPER,), jnp.int32),
            pltpu.VMEM((SUPER,), jnp.int32),
            pltpu.VMEM((SUPER,), jnp.float32),
            pltpu.VMEM((CHUNK, HALF), jnp.float32),
            pltpu.VMEM((CHUNK, HALF), jnp.float32),
            pltpu.VMEM_SHARED((N_NODES_PAD, HALF), jnp.float32),
            pltpu.SemaphoreType.DMA,
            pltpu.SemaphoreType.DMA,
            pltpu.SemaphoreType.DMA,
        ],
    )
    def prop(t0_hbm, t1_hbm, eidx_hbm, ev_hbm,
             out0_hbm, out1_hbm, sidx, didx, evb, rows0, rows1, acc,
             sem_st, sem_g, sem_s):
        c = lax.axis_index("c")
        s = lax.axis_index("s")
        r0 = pl.multiple_of(s * ROWS_PER_TILE, 8)
        rowbuf = [rows0, rows1]

        def run(table, out):
            pltpu.sync_copy(table.at[pl.ds(r0, ROWS_PER_TILE)],
                            acc.at[pl.ds(r0, ROWS_PER_TILE)])
            plsc.subcore_barrier()

            def gather(k):
                rk = rowbuf[k % 2]
                return [pltpu.async_copy(
                    table.at[sidx.at[pl.ds(k * CHUNK + j * SUB, SUB)]],
                    rk.at[pl.ds(j * SUB, SUB)], sem_g)
                    for j in range(NSUB)]

            def scatter(k):
                rk = rowbuf[k % 2]
                return [pltpu.async_copy(
                    rk.at[pl.ds(j * SUB, SUB)],
                    acc.at[didx.at[pl.ds(k * CHUNK + j * SUB, SUB)]],
                    sem_s, add=True)
                    for j in range(NSUB)]

            def scale(k):
                rk = rowbuf[k % 2]

                def scale8(i, cc):
                    for u in range(8):
                        e = i * 8 + u
                        v = plsc.load_gather(
                            evb, [jnp.full((16,), k * CHUNK + e, jnp.int32)])
                        rk[e, pl.ds(0, 16)] = rk[e, pl.ds(0, 16)] * v
                        rk[e, pl.ds(16, 16)] = rk[e, pl.ds(16, 16)] * v
                    return cc

                lax.fori_loop(0, CHUNK // 8, scale8, 0)

            def super_body(t, carry):
                base = pl.multiple_of(s * EDGES_PER_TILE + t * SUPER, 8)
                st = [
                    pltpu.async_copy(eidx_hbm.at[0, pl.ds(base, SUPER)],
                                     sidx, sem_st),
                    pltpu.async_copy(eidx_hbm.at[1, pl.ds(base, SUPER)],
                                     didx, sem_st),
                    pltpu.async_copy(ev_hbm.at[pl.ds(base, SUPER)], evb,
                                     sem_st),
                ]
                for cp in st:
                    cp.wait()
                g_h = {0: gather(0)}
                s_h = {}
                for k in range(KCH):
                    for cp in g_h[k]:
                        cp.wait()
                    if k + 1 < KCH:
                        if k >= 1:
                            for cp in s_h[k - 1]:
                                cp.wait()
                        g_h[k + 1] = gather(k + 1)
                    scale(k)
                    s_h[k] = scatter(k)
                for k in (KCH - 2, KCH - 1):
                    for cp in s_h[k]:
                        cp.wait()
                return carry

            lax.fori_loop(0, SUPERS, super_body, 0)
            plsc.subcore_barrier()
            pltpu.sync_copy(acc.at[pl.ds(r0, ROWS_PER_TILE)],
                            out.at[pl.ds(r0, ROWS_PER_TILE)])

        pl.when(c == 0)(lambda: run(t0_hbm, out0_hbm))
        pl.when(c == 1)(lambda: run(t1_hbm, out1_hbm))

    return prop(t0, t1, eidx, ev)


def kernel(edge_index, edge_vals, user_emb, item_emb, image_embedding,
           text_embedding, W_img_res, b_img_res, W_img_proj, b_img_proj,
           W_txt_res, b_txt_res, W_txt_proj, b_txt_proj, W_gate_img,
           b_gate_img, W_gate_txt, b_gate_txt, fusion_weight, res_scale):
    yi, yt = _dense_mm(
        image_embedding, text_embedding, W_img_res, W_txt_res,
        b_img_res.reshape(1, LATDIM), b_txt_res.reshape(1, LATDIM),
    )
    item_part = _gated_item(
        yi, yt, item_emb,
        W_img_proj, b_img_proj.reshape(1, LATDIM),
        W_txt_proj, b_txt_proj.reshape(1, LATDIM),
        W_gate_img, b_gate_img.reshape(1, LATDIM),
        W_gate_txt, b_gate_txt.reshape(1, LATDIM),
        fusion_weight.reshape(1, 3), res_scale.reshape(1, 1),
    )
    combined = jnp.concatenate(
        [user_emb, item_part,
         jnp.zeros((N_NODES_PAD - N_NODES, LATDIM), jnp.float32)], axis=0)
    t0 = combined[:, :HALF]
    t1 = combined[:, HALF:]
    eidx = jnp.pad(edge_index.astype(jnp.int32),
                   ((0, 0), (0, N_EDGES_PAD - N_EDGES)))
    ev = jnp.pad(edge_vals, (0, N_EDGES_PAD - N_EDGES))
    out0, out1 = _sc_propagate(t0, t1, eidx, ev)
    return jnp.concatenate([out0, out1], axis=1)[:N_NODES]


# speedup vs baseline: 1.3394x; 1.3394x over previous
import functools

import jax
import jax.numpy as jnp
from jax import lax
from jax.experimental import pallas as pl
from jax.experimental.pallas import tpu as pltpu
from jax.experimental.pallas import tpu_sc as plsc

N_USER = 40000
N_ITEM = 10000
N_NODES = 50000
N_EDGES = 800000
LATDIM = 64
HALF = 32
IMG_DIM = 4096
TXT_DIM = 384

ROW_BLK = 1000

NUM_TILES = 16
EDGES_PER_TILE = N_EDGES // NUM_TILES
CHUNK = 200
SUB = 100
NSUB = CHUNK // SUB
KCH = 10
SUPER = KCH * CHUNK
SUPERS = EDGES_PER_TILE // SUPER
N_NODES_PAD = 50048
ROWS_PER_TILE = N_NODES_PAD // NUM_TILES


def _mm_body(img_ref, txt_ref, wi_ref, wt_ref, bi_ref, bt_ref, yi_ref, yt_ref):
    yi_ref[...] = (
        jnp.dot(img_ref[...], wi_ref[...], preferred_element_type=jnp.float32)
        + bi_ref[...]
    )
    yt_ref[...] = (
        jnp.dot(txt_ref[...], wt_ref[...], preferred_element_type=jnp.float32)
        + bt_ref[...]
    )


def _dense_mm(img, txt, wi, wt, bi, bt):
    grid = (N_ITEM // ROW_BLK,)
    return pl.pallas_call(
        _mm_body,
        grid=grid,
        in_specs=[
            pl.BlockSpec((ROW_BLK, IMG_DIM), lambda i: (i, 0)),
            pl.BlockSpec((ROW_BLK, TXT_DIM), lambda i: (i, 0)),
            pl.BlockSpec((IMG_DIM, LATDIM), lambda i: (0, 0)),
            pl.BlockSpec((TXT_DIM, LATDIM), lambda i: (0, 0)),
            pl.BlockSpec((1, LATDIM), lambda i: (0, 0)),
            pl.BlockSpec((1, LATDIM), lambda i: (0, 0)),
        ],
        out_specs=[
            pl.BlockSpec((ROW_BLK, LATDIM), lambda i: (i, 0)),
            pl.BlockSpec((ROW_BLK, LATDIM), lambda i: (i, 0)),
        ],
        out_shape=[
            jax.ShapeDtypeStruct((N_ITEM, LATDIM), jnp.float32),
            jax.ShapeDtypeStruct((N_ITEM, LATDIM), jnp.float32),
        ],
    )(img, txt, wi, wt, bi, bt)


def _bn(x):
    m = jnp.mean(x, axis=0, keepdims=True)
    d = x - m
    v = jnp.mean(d * d, axis=0, keepdims=True)
    return d * lax.rsqrt(v + 1e-5)


def _leaky(x):
    return jnp.where(x >= 0, x, 0.2 * x)


def _gates_body(
    yi_ref, yt_ref, item_ref, wip_ref, bip_ref, wtp_ref, btp_ref,
    wgi_ref, bgi_ref, wgt_ref, bgt_ref, fw_ref, rs_ref, out_ref,
):
    rs = rs_ref[0, 0]
    x_img = _leaky(_bn(yi_ref[...]))
    img_feat = rs * x_img + _leaky(
        _bn(jnp.dot(x_img, wip_ref[...], preferred_element_type=jnp.float32)
            + bip_ref[...])
    )
    x_txt = _leaky(_bn(yt_ref[...]))
    txt_feat = rs * x_txt + _leaky(
        _bn(jnp.dot(x_txt, wtp_ref[...], preferred_element_type=jnp.float32)
            + btp_ref[...])
    )
    gate_img = jax.nn.sigmoid(
        _bn(jnp.dot(img_feat, wgi_ref[...], preferred_element_type=jnp.float32)
            + bgi_ref[...])
    )
    gate_txt = jax.nn.sigmoid(
        _bn(jnp.dot(txt_feat, wgt_ref[...], preferred_element_type=jnp.float32)
            + bgt_ref[...])
    )
    f0 = fw_ref[0, 0]
    f1 = fw_ref[0, 1]
    f2 = fw_ref[0, 2]
    fm = jnp.maximum(f0, jnp.maximum(f1, f2))
    e0 = jnp.exp(f0 - fm)
    e1 = jnp.exp(f1 - fm)
    e2 = jnp.exp(f2 - fm)
    es = e0 + e1 + e2
    w0 = e0 / es
    w1 = e1 / es
    w2 = e2 / es
    out_ref[...] = item_ref[...] * (w0 + w1 * gate_img + w2 * gate_txt)


def _gated_item(yi, yt, item, wip, bip, wtp, btp, wgi, bgi, wgt, bgt, fw, rs):
    return pl.pallas_call(
        _gates_body,
        out_shape=jax.ShapeDtypeStruct((N_ITEM, LATDIM), jnp.float32),
    )(yi, yt, item, wip, bip, wtp, btp, wgi, bgi, wgt, bgt, fw, rs)


def _sc_propagate(t0, t1, eidx, ev):
    mesh = plsc.VectorSubcoreMesh(core_axis_name="c", subcore_axis_name="s")

    @functools.partial(
        pl.kernel,
        out_type=[
            jax.ShapeDtypeStruct((N_NODES_PAD, HALF), jnp.float32),
            jax.ShapeDtypeStruct((N_NODES_PAD, HALF), jnp.float32),
        ],
        mesh=mesh,
        compiler_params=pltpu.CompilerParams(
            needs_layout_passes=False, use_tc_tiling_on_sc=False),
        scratch_types=[
            pltpu.VMEM((KCH * NSUB, SUB), jnp.int32),
            pltpu.VMEM((KCH * NSUB, SUB), jnp.int32),
            pltpu.VMEM((SUPER,), jnp.float32),
            pltpu.VMEM((CHUNK, HALF), jnp.float32),
            pltpu.VMEM((CHUNK, HALF), jnp.float32),
            pltpu.VMEM_SHARED((N_NODES_PAD, HALF), jnp.float32),
            pltpu.SemaphoreType.DMA,
            pltpu.SemaphoreType.DMA,
            pltpu.SemaphoreType.DMA,
        ],
    )
    def prop(t0_hbm, t1_hbm, eidx_hbm, ev_hbm,
             out0_hbm, out1_hbm, sidx, didx, evb, rows0, rows1, acc,
             sem_st, sem_g, sem_s):
        c = lax.axis_index("c")
        s = lax.axis_index("s")
        r0 = pl.multiple_of(s * ROWS_PER_TILE, 8)
        rowbuf = [rows0, rows1]

        def run(table, out):
            pltpu.sync_copy(table.at[pl.ds(r0, ROWS_PER_TILE)],
                            acc.at[pl.ds(r0, ROWS_PER_TILE)])
            plsc.subcore_barrier()

            def gather(k):
                rk = rowbuf[k % 2]
                return [pltpu.async_copy(
                    table.at[sidx.at[k * NSUB + j]],
                    rk.at[pl.ds(j * SUB, SUB)], sem_g)
                    for j in range(NSUB)]

            def scatter(k):
                rk = rowbuf[k % 2]
                return [pltpu.async_copy(
                    rk.at[pl.ds(j * SUB, SUB)],
                    acc.at[didx.at[k * NSUB + j]],
                    sem_s, add=True)
                    for j in range(NSUB)]

            def scale(k):
                rk = rowbuf[k % 2]

                def scale8(i, cc):
                    for u in range(8):
                        e = i * 8 + u
                        v = plsc.load_gather(
                            evb, [jnp.full((16,), k * CHUNK + e, jnp.int32)])
                        rk[e, pl.ds(0, 16)] = rk[e, pl.ds(0, 16)] * v
                        rk[e, pl.ds(16, 16)] = rk[e, pl.ds(16, 16)] * v
                    return cc

                lax.fori_loop(0, CHUNK // 8, scale8, 0)

            def super_body(t, carry):
                base = pl.multiple_of(s * EDGES_PER_TILE + t * SUPER, 8)
                idx_row0 = pl.multiple_of(
                    (s * EDGES_PER_TILE + t * SUPER) // SUB, 4)
                st = [
                    pltpu.async_copy(
                        eidx_hbm.at[0, pl.ds(idx_row0, KCH * NSUB)],
                        sidx, sem_st),
                    pltpu.async_copy(
                        eidx_hbm.at[1, pl.ds(idx_row0, KCH * NSUB)],
                        didx, sem_st),
                    pltpu.async_copy(ev_hbm.at[pl.ds(base, SUPER)], evb,
                                     sem_st),
                ]
                for cp in st:
                    cp.wait()
                g_h = {0: gather(0)}
                s_h = {}
                for k in range(KCH):
                    for cp in g_h[k]:
                        cp.wait()
                    if k + 1 < KCH:
                        if k >= 1:
                            for cp in s_h[k - 1]:
                                cp.wait()
                        g_h[k + 1] = gather(k + 1)
                    scale(k)
                    s_h[k] = scatter(k)
                for k in (KCH - 2, KCH - 1):
                    for cp in s_h[k]:
                        cp.wait()
                return carry

            lax.fori_loop(0, SUPERS, super_body, 0)
            plsc.subcore_barrier()
            pltpu.sync_copy(acc.at[pl.ds(r0, ROWS_PER_TILE)],
                            out.at[pl.ds(r0, ROWS_PER_TILE)])

        pl.when(c == 0)(lambda: run(t0_hbm, out0_hbm))
        pl.when(c == 1)(lambda: run(t1_hbm, out1_hbm))

    return prop(t0, t1, eidx, ev)


def kernel(edge_index, edge_vals, user_emb, item_emb, image_embedding,
           text_embedding, W_img_res, b_img_res, W_img_proj, b_img_proj,
           W_txt_res, b_txt_res, W_txt_proj, b_txt_proj, W_gate_img,
           b_gate_img, W_gate_txt, b_gate_txt, fusion_weight, res_scale):
    yi, yt = _dense_mm(
        image_embedding, text_embedding, W_img_res, W_txt_res,
        b_img_res.reshape(1, LATDIM), b_txt_res.reshape(1, LATDIM),
    )
    item_part = _gated_item(
        yi, yt, item_emb,
        W_img_proj, b_img_proj.reshape(1, LATDIM),
        W_txt_proj, b_txt_proj.reshape(1, LATDIM),
        W_gate_img, b_gate_img.reshape(1, LATDIM),
        W_gate_txt, b_gate_txt.reshape(1, LATDIM),
        fusion_weight.reshape(1, 3), res_scale.reshape(1, 1),
    )
    combined = jnp.concatenate(
        [user_emb, item_part,
         jnp.zeros((N_NODES_PAD - N_NODES, LATDIM), jnp.float32)], axis=0)
    t0 = combined[:, :HALF]
    t1 = combined[:, HALF:]
    eidx = edge_index.astype(jnp.int32).reshape(2, N_EDGES // SUB, SUB)
    out0, out1 = _sc_propagate(t0, t1, eidx, edge_vals)
    return jnp.concatenate([out0, out1], axis=1)[:N_NODES]
